# Initial kernel scaffold; baseline (speedup 1.0000x reference)
import jax
import jax.numpy as jnp
from jax.experimental import pallas as pl


def kernel(label, tables):
    raise NotImplementedError("write your pallas kernel here")


# speedup vs baseline: 4.0561x; 4.0561x over previous
import functools

import jax
import jax.numpy as jnp
from jax import lax
from jax.experimental import pallas as pl
from jax.experimental.pallas import tpu as pltpu
from jax.experimental.pallas import tpu_sc as plsc

_F = 26
_V1 = 1001
_D = 128
_B = 16384
_R = _B * _F
_NC = 2
_NS = 16
_NW = _NC * _NS
_RPW = _R // _NW
_C = 128
_NCHUNK = _RPW // _C
_L = 16

_mesh = plsc.VectorSubcoreMesh(core_axis_name="c", subcore_axis_name="s")


@functools.partial(
    pl.kernel,
    mesh=_mesh,
    out_type=jax.ShapeDtypeStruct((_R, _D), jnp.float32),
    scratch_types=[
        pltpu.VMEM((_RPW,), jnp.int32),
        pltpu.VMEM((_C, _D), jnp.float32),
        pltpu.VMEM((_C, _D), jnp.float32),
        pltpu.SemaphoreType.DMA,
        pltpu.SemaphoreType.DMA,
    ],
)
def _gather_kernel(tab_hbm, lab_hbm, out_hbm, idx_v, buf0, buf1, sem0, sem1):
    wid = lax.axis_index("s") * _NC + lax.axis_index("c")
    base = wid * _RPW
    pltpu.sync_copy(lab_hbm.at[pl.ds(base, _RPW)], idx_v)

    lanes = lax.iota(jnp.int32, _L)

    def idx_body(k, carry):
        off = k * _L
        lv = idx_v[pl.ds(off, _L)]
        field = ((base + off) + lanes) % _F
        g = field * _V1 + jnp.where(lv == -1, _V1 - 1, lv)
        idx_v[pl.ds(off, _L)] = g
        return carry

    lax.fori_loop(0, _RPW // _L, idx_body, 0)

    def chunk_body(p, carry):
        c0 = p * 2
        c1 = c0 + 1
        cp0 = pltpu.async_copy(tab_hbm.at[idx_v.at[pl.ds(c0 * _C, _C)]], buf0, sem0)
        cp1 = pltpu.async_copy(tab_hbm.at[idx_v.at[pl.ds(c1 * _C, _C)]], buf1, sem1)
        cp0.wait()
        pltpu.sync_copy(buf0, out_hbm.at[pl.ds(base + c0 * _C, _C)])
        cp1.wait()
        pltpu.sync_copy(buf1, out_hbm.at[pl.ds(base + c1 * _C, _C)])
        return carry

    lax.fori_loop(0, _NCHUNK // 2, chunk_body, 0)


def kernel(label, tables):
    lab_flat = label.reshape(_R)
    tab_flat = tables.reshape(_F * _V1, _D)
    out = _gather_kernel(tab_flat, lab_flat)
    return out.reshape(_B, _F * _D)


# speedup vs baseline: 4.3661x; 1.0764x over previous
import functools

import jax
import jax.numpy as jnp
from jax import lax
from jax.experimental import pallas as pl
from jax.experimental.pallas import tpu as pltpu
from jax.experimental.pallas import tpu_sc as plsc

_F = 26
_V1 = 1001
_D = 128
_B = 16384
_R = _B * _F
_NC = 2
_NS = 16
_NW = _NC * _NS
_RPW = _R // _NW
_C = 128
_NCHUNK = _RPW // _C
_L = 16

_mesh = plsc.VectorSubcoreMesh(core_axis_name="c", subcore_axis_name="s")


_NBUF = 4
_LAG = 2
_NG = _NCHUNK // _NBUF


@functools.partial(
    pl.kernel,
    mesh=_mesh,
    out_type=jax.ShapeDtypeStruct((_R, _D), jnp.float32),
    scratch_types=[
        pltpu.VMEM((_RPW,), jnp.int32),
        pltpu.VMEM((_C, _D), jnp.float32),
        pltpu.VMEM((_C, _D), jnp.float32),
        pltpu.VMEM((_C, _D), jnp.float32),
        pltpu.VMEM((_C, _D), jnp.float32),
        pltpu.SemaphoreType.DMA,
        pltpu.SemaphoreType.DMA,
        pltpu.SemaphoreType.DMA,
        pltpu.SemaphoreType.DMA,
        pltpu.SemaphoreType.DMA,
        pltpu.SemaphoreType.DMA,
        pltpu.SemaphoreType.DMA,
        pltpu.SemaphoreType.DMA,
    ],
)
def _gather_kernel(tab_hbm, lab_hbm, out_hbm, idx_v,
                   b0, b1, b2, b3, g0, g1, g2, g3, o0, o1, o2, o3):
    bufs = (b0, b1, b2, b3)
    gsem = (g0, g1, g2, g3)
    osem = (o0, o1, o2, o3)

    wid = lax.axis_index("s") * _NC + lax.axis_index("c")
    base = wid * _RPW
    pltpu.sync_copy(lab_hbm.at[pl.ds(base, _RPW)], idx_v)

    lanes = lax.iota(jnp.int32, _L)

    def compute_idx(c):
        for k in range(_C // _L):
            off = c * _C + k * _L
            lv = idx_v[pl.ds(off, _L)]
            field = ((base + off) + lanes) % _F
            g = field * _V1 + jnp.where(lv == -1, _V1 - 1, lv)
            idx_v[pl.ds(off, _L)] = g

    def gather_copy(c, j):
        return pltpu.make_async_copy(
            tab_hbm.at[idx_v.at[pl.ds(c * _C, _C)]], bufs[j], gsem[j])

    def out_copy(c, j):
        return pltpu.make_async_copy(
            bufs[j], out_hbm.at[pl.ds(base + c * _C, _C)], osem[j])

    for j in range(_NBUF):
        compute_idx(j)
        gather_copy(j, j).start()
        if j >= _LAG:
            j2 = j - _LAG
            gather_copy(j2, j2).wait()
            out_copy(j2, j2).start()

    def group_body(p, carry):
        for j in range(_NBUF):
            c = p * _NBUF + j
            compute_idx(c)
            out_copy(c - _NBUF, j).wait()
            gather_copy(c, j).start()
            j2 = (j - _LAG) % _NBUF
            gather_copy(c - _LAG, j2).wait()
            out_copy(c - _LAG, j2).start()
        return carry

    lax.fori_loop(1, _NG, group_body, 0)

    last = _NCHUNK - _LAG
    for i in range(_LAG):
        c = last + i
        gather_copy(c, c % _NBUF).wait()
        out_copy(c, c % _NBUF).start()
    for j in range(_NBUF):
        out_copy(_NCHUNK - _NBUF + j, j).wait()


def kernel(label, tables):
    lab_flat = label.reshape(_R)
    tab_flat = tables.reshape(_F * _V1, _D)
    out = _gather_kernel(tab_flat, lab_flat)
    return out.reshape(_B, _F * _D)


# speedup vs baseline: 9.2329x; 2.1146x over previous
import functools

import jax
import jax.numpy as jnp
from jax import lax
from jax.experimental import pallas as pl
from jax.experimental.pallas import tpu as pltpu
from jax.experimental.pallas import tpu_sc as plsc

_F = 26
_V1 = 1001
_D = 128
_B = 16384
_R = _B * _F
_NC = 2
_NS = 16
_NW = _NC * _NS
_RPW = _R // _NW
_BPW = _B // _NW
_C = 104
_CB = _C // _F
_NCHUNK = _RPW // _C
_L = 16

_NBUF = 4
_LAG = 2
_NG = _NCHUNK // _NBUF
_GROUP_SLICES = _NBUF * _C // _L

_mesh = plsc.VectorSubcoreMesh(core_axis_name="c", subcore_axis_name="s")


@functools.partial(
    pl.kernel,
    mesh=_mesh,
    out_type=jax.ShapeDtypeStruct((_B, _F * _D), jnp.float32),
    scratch_types=[
        pltpu.VMEM((_RPW,), jnp.int32),
        pltpu.VMEM((_C, _D), jnp.float32),
        pltpu.VMEM((_C, _D), jnp.float32),
        pltpu.VMEM((_C, _D), jnp.float32),
        pltpu.VMEM((_C, _D), jnp.float32),
        pltpu.SemaphoreType.DMA,
        pltpu.SemaphoreType.DMA,
        pltpu.SemaphoreType.DMA,
        pltpu.SemaphoreType.DMA,
        pltpu.SemaphoreType.DMA,
        pltpu.SemaphoreType.DMA,
        pltpu.SemaphoreType.DMA,
        pltpu.SemaphoreType.DMA,
    ],
)
def _gather_kernel(tab_hbm, lab_hbm, out_hbm, idx_v,
                   b0, b1, b2, b3, g0, g1, g2, g3, o0, o1, o2, o3):
    bufs = (b0, b1, b2, b3)
    gsem = (g0, g1, g2, g3)
    osem = (o0, o1, o2, o3)

    wid = lax.axis_index("s") * _NC + lax.axis_index("c")
    base = wid * _RPW
    bbase = wid * _BPW
    pltpu.sync_copy(lab_hbm.at[pl.ds(base, _RPW)], idx_v)

    lanes = lax.iota(jnp.int32, _L)

    def compute_idx_group(p):
        for k in range(_GROUP_SLICES):
            off = p * (_NBUF * _C) + k * _L
            lv = idx_v[pl.ds(off, _L)]
            field = ((base + off) + lanes) % _F
            g = field * _V1 + jnp.where(lv == -1, _V1 - 1, lv)
            idx_v[pl.ds(off, _L)] = g

    def gather_copy(c, j):
        return pltpu.make_async_copy(
            tab_hbm.at[idx_v.at[pl.ds(c * _C, _C)]], bufs[j], gsem[j])

    def out_copy(c, j):
        return pltpu.make_async_copy(
            bufs[j].reshape(_CB, _F * _D),
            out_hbm.at[pl.ds(bbase + c * _CB, _CB)], osem[j])

    compute_idx_group(0)
    for j in range(_NBUF):
        gather_copy(j, j).start()
        if j >= _LAG:
            j2 = j - _LAG
            gather_copy(j2, j2).wait()
            out_copy(j2, j2).start()

    def group_body(p, carry):
        compute_idx_group(p)
        for j in range(_NBUF):
            c = p * _NBUF + j
            out_copy(c - _NBUF, j).wait()
            gather_copy(c, j).start()
            j2 = (j - _LAG) % _NBUF
            gather_copy(c - _LAG, j2).wait()
            out_copy(c - _LAG, j2).start()
        return carry

    lax.fori_loop(1, _NG, group_body, 0)

    last = _NCHUNK - _LAG
    for i in range(_LAG):
        c = last + i
        gather_copy(c, c % _NBUF).wait()
        out_copy(c, c % _NBUF).start()
    for j in range(_NBUF):
        out_copy(_NCHUNK - _NBUF + j, j).wait()


def kernel(label, tables):
    lab_flat = label.reshape(_R)
    tab_flat = tables.reshape(_F * _V1, _D)
    return _gather_kernel(tab_flat, lab_flat)


# speedup vs baseline: 9.2520x; 1.0021x over previous
import functools

import jax
import jax.numpy as jnp
from jax import lax
from jax.experimental import pallas as pl
from jax.experimental.pallas import tpu as pltpu
from jax.experimental.pallas import tpu_sc as plsc

_F = 26
_V1 = 1001
_D = 128
_B = 16384
_R = _B * _F
_NC = 2
_NS = 16
_NW = _NC * _NS
_RPW = _R // _NW
_BPW = _B // _NW
_C = 104
_CB = _C // _F
_NCHUNK = _RPW // _C
_L = 16

_NBUF = 8
_LAG = 4
_NG = _NCHUNK // _NBUF
_GROUP_SLICES = _NBUF * _C // _L

_mesh = plsc.VectorSubcoreMesh(core_axis_name="c", subcore_axis_name="s")


@functools.partial(
    pl.kernel,
    mesh=_mesh,
    out_type=jax.ShapeDtypeStruct((_B, _F * _D), jnp.float32),
    scratch_types=[
        pltpu.VMEM((_RPW,), jnp.int32),
    ] + [pltpu.VMEM((_C, _D), jnp.float32)] * _NBUF
      + [pltpu.SemaphoreType.DMA] * (2 * _NBUF),
)
def _gather_kernel(tab_hbm, lab_hbm, out_hbm, idx_v, *bufs_and_sems):
    bufs = bufs_and_sems[:_NBUF]
    gsem = bufs_and_sems[_NBUF:2 * _NBUF]
    osem = bufs_and_sems[2 * _NBUF:]

    wid = lax.axis_index("s") * _NC + lax.axis_index("c")
    base = wid * _RPW
    bbase = wid * _BPW
    pltpu.sync_copy(lab_hbm.at[pl.ds(base, _RPW)], idx_v)

    lanes = lax.iota(jnp.int32, _L)

    def compute_idx_group(p):
        for k in range(_GROUP_SLICES):
            off = p * (_NBUF * _C) + k * _L
            lv = idx_v[pl.ds(off, _L)]
            field = ((base + off) + lanes) % _F
            g = field * _V1 + jnp.where(lv == -1, _V1 - 1, lv)
            idx_v[pl.ds(off, _L)] = g

    def gather_copy(c, j):
        return pltpu.make_async_copy(
            tab_hbm.at[idx_v.at[pl.ds(c * _C, _C)]], bufs[j], gsem[j])

    def out_copy(c, j):
        return pltpu.make_async_copy(
            bufs[j].reshape(_CB, _F * _D),
            out_hbm.at[pl.ds(bbase + c * _CB, _CB)], osem[j])

    compute_idx_group(0)
    for j in range(_NBUF):
        gather_copy(j, j).start()
        if j >= _LAG:
            j2 = j - _LAG
            gather_copy(j2, j2).wait()
            out_copy(j2, j2).start()

    def group_body(p, carry):
        compute_idx_group(p)
        for j in range(_NBUF):
            c = p * _NBUF + j
            out_copy(c - _NBUF, j).wait()
            gather_copy(c, j).start()
            j2 = (j - _LAG) % _NBUF
            gather_copy(c - _LAG, j2).wait()
            out_copy(c - _LAG, j2).start()
        return carry

    lax.fori_loop(1, _NG, group_body, 0)

    last = _NCHUNK - _LAG
    for i in range(_LAG):
        c = last + i
        gather_copy(c, c % _NBUF).wait()
        out_copy(c, c % _NBUF).start()
    for j in range(_NBUF):
        out_copy(_NCHUNK - _NBUF + j, j).wait()


def kernel(label, tables):
    lab_flat = label.reshape(_R)
    tab_flat = tables.reshape(_F * _V1, _D)
    return _gather_kernel(tab_flat, lab_flat)
